# baseline (speedup 1.0000x reference)
import functools

import jax
import jax.numpy as jnp
from jax import lax
from jax.experimental import pallas as pl
from jax.experimental.pallas import tpu as pltpu
from jax.experimental.pallas import tpu_sc as plsc

_NUM_WORKERS = 32
_STEPS = 64
_NBUF = 2
_TC_BATCH_BLOCK = 512
_PAIR_SPLIT = 13


def _make_gather(n_rows, embed_dim, dtype):
    mesh = plsc.VectorSubcoreMesh(core_axis_name="c", subcore_axis_name="s")
    per_w = n_rows // _NUM_WORKERS
    ch = per_w // _STEPS

    @functools.partial(
        pl.kernel,
        out_type=jax.ShapeDtypeStruct((n_rows, embed_dim), dtype),
        mesh=mesh,
        compiler_params=pltpu.CompilerParams(use_tc_tiling_on_sc=False),
        scratch_types=[
            pltpu.VMEM((1, per_w), jnp.int32),
            pltpu.VMEM((_NBUF, ch, embed_dim), jnp.float32),
            pltpu.SemaphoreType.DMA,
            pltpu.SemaphoreType.DMA,
            pltpu.SemaphoreType.DMA,
            pltpu.SemaphoreType.DMA,
        ],
    )
    def gather_kernel(table_hbm, idx_hbm, out_hbm, idx_v, rows_v, g0, g1, w0, w1):
        gsem = [g0, g1]
        wsem = [w0, w1]
        wid = lax.axis_index("s") * 2 + lax.axis_index("c")
        r0 = wid * per_w

        pltpu.sync_copy(idx_hbm.at[0, pl.ds(r0, per_w)], idx_v.at[0])

        def issue_gather(k, slot):
            pltpu.async_copy(
                table_hbm.at[idx_v.at[0, pl.ds(k * ch, ch)]],
                rows_v.at[slot],
                gsem[slot],
            )

        def wait_gather(slot):
            pltpu.make_async_copy(
                table_hbm.at[pl.ds(0, ch)], rows_v.at[slot], gsem[slot]
            ).wait()

        def issue_writes(k, slot):
            pltpu.async_copy(
                rows_v.at[slot],
                out_hbm.at[pl.ds(r0 + k * ch, ch)],
                wsem[slot],
            )

        def drain_writes(slot):
            pltpu.make_async_copy(
                rows_v.at[slot], out_hbm.at[pl.ds(0, ch)], wsem[slot]
            ).wait()

        issue_gather(0, 0)

        @pl.loop(0, _STEPS, step=_NBUF)
        def _(t):
            for b in range(_NBUF):
                k = t + b
                nslot = (b + 1) % _NBUF

                @pl.when(k + 1 < _STEPS)
                def _prefetch():
                    @pl.when(k + 1 >= _NBUF)
                    def _drain():
                        drain_writes(nslot)

                    issue_gather(k + 1, nslot)

                wait_gather(b)
                issue_writes(k, b)

        for b in range(_NBUF):
            drain_writes(b)

    return gather_kernel


def kernel(style_idx, table):
    batch, hist = style_idx.shape
    num_rows, embed_dim = table.shape
    row = hist * embed_dim
    pairs = row // 128
    bblk = _TC_BATCH_BLOCK
    jb = batch // bblk
    chunks = [(0, _PAIR_SPLIT), (_PAIR_SPLIT, pairs)]

    xt = None
    for r_lo, r_hi in chunks:
        rc = r_hi - r_lo

        idx_c = (
            style_idx[:, 2 * r_lo : 2 * r_hi]
            .reshape(batch, rc, 2)
            .transpose(1, 0, 2)
            .reshape(1, batch * rc * 2)
            .astype(jnp.int32)
        )
        g = _make_gather(batch * rc * 2, embed_dim, table.dtype)(table, idx_c)

        in2d = g.reshape(batch * rc, 128)

        def transpose_body(*refs):
            in_ref, out_ref = refs[0], refs[-1]
            out_ref[...] = in_ref[...].T

        operands = [in2d] if xt is None else [in2d, xt]
        in_specs = [
            pl.BlockSpec((bblk, 128), lambda j, r, _jb=jb: (r * _jb + j, 0)),
        ]
        if xt is not None:
            in_specs.append(pl.BlockSpec(memory_space=pltpu.MemorySpace.HBM))
        xt = pl.pallas_call(
            transpose_body,
            out_shape=jax.ShapeDtypeStruct((row, batch), table.dtype),
            grid=(jb, rc),
            in_specs=in_specs,
            out_specs=pl.BlockSpec(
                (128, bblk), lambda j, r, _lo=r_lo: (_lo + r, j)
            ),
            input_output_aliases={} if len(operands) == 1 else {1: 0},
            compiler_params=pltpu.CompilerParams(
                dimension_semantics=("parallel", "arbitrary")
            ),
        )(*operands)

    x3 = xt.reshape(hist, embed_dim, batch)
    return jnp.transpose(x3, (2, 0, 1))


# speedup vs baseline: 2.3246x; 2.3246x over previous
import functools

import jax
import jax.numpy as jnp
from jax import lax
from jax.experimental import pallas as pl
from jax.experimental.pallas import tpu as pltpu
from jax.experimental.pallas import tpu_sc as plsc

_NUM_WORKERS = 32
_NBUF = 2
_TC_BATCH_BLOCK = 512
_NCHUNKS = 5
_CH = 256


def _make_gather(batch, hist, embed_dim, r_lo, rc, dtype):
    mesh = plsc.VectorSubcoreMesh(core_axis_name="c", subcore_axis_name="s")
    pb = batch // _NUM_WORKERS
    rows_r = 2 * pb
    spr = rows_r // _CH
    steps = rc * spr
    n_rows = rc * batch * 2

    @functools.partial(
        pl.kernel,
        out_type=jax.ShapeDtypeStruct((n_rows, embed_dim), dtype),
        mesh=mesh,
        compiler_params=pltpu.CompilerParams(
            use_tc_tiling_on_sc=False, needs_layout_passes=False
        ),
        scratch_types=[
            pltpu.VMEM((pb * hist,), jnp.int32),
            pltpu.VMEM((rc * rows_r,), jnp.int32),
            pltpu.VMEM((_NBUF, _CH, embed_dim), jnp.float32),
            pltpu.SemaphoreType.DMA,
            pltpu.SemaphoreType.DMA,
            pltpu.SemaphoreType.DMA,
            pltpu.SemaphoreType.DMA,
        ],
    )
    def gather_kernel(
        table_hbm, idx_hbm, out_hbm, idx_raw, idx_v, rows_v, g0, g1, w0, w1
    ):
        gsem = [g0, g1]
        wsem = [w0, w1]
        wid = lax.axis_index("s") * 2 + lax.axis_index("c")
        b0 = wid * pb

        pltpu.sync_copy(idx_hbm.at[0, pl.ds(b0 * hist, pb * hist)], idx_raw)

        lane = jnp.arange(16, dtype=jnp.int32)
        b_of = lane // 2
        hh_of = lane - 2 * b_of

        @pl.loop(0, rc * rows_r // 16)
        def _(gidx):
            r = gidx // (rows_r // 16)
            bg = gidx - r * (rows_r // 16)
            src = (bg * 8 + b_of) * hist + 2 * (r_lo + r) + hh_of
            idx_v[pl.ds(gidx * 16, 16)] = plsc.load_gather(idx_raw, [src])

        def out_off(k):
            r = k // spr
            sub = k - r * spr
            return r * batch * 2 + b0 * 2 + sub * _CH

        def issue_gather(k, slot):
            pltpu.async_copy(
                table_hbm.at[idx_v.at[pl.ds(k * _CH, _CH)]],
                rows_v.at[slot],
                gsem[slot],
            )

        def wait_gather(slot):
            pltpu.make_async_copy(
                table_hbm.at[pl.ds(0, _CH)], rows_v.at[slot], gsem[slot]
            ).wait()

        def issue_writes(k, slot):
            pltpu.async_copy(
                rows_v.at[slot],
                out_hbm.at[pl.ds(out_off(k), _CH)],
                wsem[slot],
            )

        def drain_writes(slot):
            pltpu.make_async_copy(
                rows_v.at[slot], out_hbm.at[pl.ds(0, _CH)], wsem[slot]
            ).wait()

        issue_gather(0, 0)

        @pl.loop(0, steps, step=_NBUF)
        def _(t):
            for b in range(_NBUF):
                k = t + b
                nslot = (b + 1) % _NBUF

                @pl.when(k + 1 < steps)
                def _prefetch():
                    @pl.when(k + 1 >= _NBUF)
                    def _drain():
                        drain_writes(nslot)

                    issue_gather(k + 1, nslot)

                wait_gather(b)
                issue_writes(k, b)

        for b in range(_NBUF):
            drain_writes(b)

    return gather_kernel


def kernel(style_idx, table):
    batch, hist = style_idx.shape
    num_rows, embed_dim = table.shape
    pairs = hist * embed_dim // 128
    rc = pairs // _NCHUNKS
    bblk = _TC_BATCH_BLOCK
    jb = batch // bblk

    idx = style_idx.reshape(1, batch * hist).astype(jnp.int32)

    xt = None
    for ci in range(_NCHUNKS):
        g = _make_gather(batch, hist, embed_dim, ci * rc, rc, table.dtype)(
            table, idx
        )

        in3 = g.reshape(rc, batch, 128)

        def transpose_body(*refs):
            in_ref, out_ref = refs[0], refs[-1]
            for r in range(rc):
                out_ref[r] = in_ref[r].T

        operands = [in3] if xt is None else [in3, xt]
        in_specs = [
            pl.BlockSpec((rc, bblk, 128), lambda j: (0, j, 0)),
        ]
        if xt is not None:
            in_specs.append(pl.BlockSpec(memory_space=pltpu.MemorySpace.HBM))
        xt = pl.pallas_call(
            transpose_body,
            out_shape=jax.ShapeDtypeStruct((pairs, 128, batch), table.dtype),
            grid=(jb,),
            in_specs=in_specs,
            out_specs=pl.BlockSpec(
                (rc, 128, bblk), lambda j, _ci=ci: (_ci, 0, j)
            ),
            input_output_aliases={} if len(operands) == 1 else {1: 0},
            compiler_params=pltpu.CompilerParams(
                dimension_semantics=("arbitrary",)
            ),
        )(*operands)

    x3 = xt.reshape(hist, embed_dim, batch)
    return jnp.transpose(x3, (2, 0, 1))


# speedup vs baseline: 2.4789x; 1.0664x over previous
import functools

import jax
import jax.numpy as jnp
from jax import lax
from jax.experimental import pallas as pl
from jax.experimental.pallas import tpu as pltpu
from jax.experimental.pallas import tpu_sc as plsc

_NUM_WORKERS = 32
_NBUF = 2
_TC_BATCH_BLOCK = 2048
_NCHUNKS = 5
_CH = 256


def _make_gather(batch, hist, embed_dim, r_lo, rc, dtype):
    mesh = plsc.VectorSubcoreMesh(core_axis_name="c", subcore_axis_name="s")
    pb = batch // _NUM_WORKERS
    rows_r = 2 * pb
    spr = rows_r // _CH
    steps = rc * spr
    n_rows = rc * batch * 2

    @functools.partial(
        pl.kernel,
        out_type=jax.ShapeDtypeStruct((n_rows, embed_dim), dtype),
        mesh=mesh,
        compiler_params=pltpu.CompilerParams(
            use_tc_tiling_on_sc=False, needs_layout_passes=False
        ),
        scratch_types=[
            pltpu.VMEM((pb * hist,), jnp.int32),
            pltpu.VMEM((rc * rows_r,), jnp.int32),
            pltpu.VMEM((_NBUF, _CH, embed_dim), jnp.float32),
            pltpu.SemaphoreType.DMA,
            pltpu.SemaphoreType.DMA,
            pltpu.SemaphoreType.DMA,
            pltpu.SemaphoreType.DMA,
        ],
    )
    def gather_kernel(
        table_hbm, idx_hbm, out_hbm, idx_raw, idx_v, rows_v, g0, g1, w0, w1
    ):
        gsem = [g0, g1]
        wsem = [w0, w1]
        wid = lax.axis_index("s") * 2 + lax.axis_index("c")
        b0 = wid * pb

        pltpu.sync_copy(idx_hbm.at[0, pl.ds(b0 * hist, pb * hist)], idx_raw)

        lane = jnp.arange(16, dtype=jnp.int32)
        b_of = lane // 2
        hh_of = lane - 2 * b_of

        @pl.loop(0, rc * rows_r // 16)
        def _(gidx):
            r = gidx // (rows_r // 16)
            bg = gidx - r * (rows_r // 16)
            src = (bg * 8 + b_of) * hist + 2 * (r_lo + r) + hh_of
            idx_v[pl.ds(gidx * 16, 16)] = plsc.load_gather(idx_raw, [src])

        def out_off(k):
            r = k // spr
            sub = k - r * spr
            return r * batch * 2 + b0 * 2 + sub * _CH

        def issue_gather(k, slot):
            pltpu.async_copy(
                table_hbm.at[idx_v.at[pl.ds(k * _CH, _CH)]],
                rows_v.at[slot],
                gsem[slot],
            )

        def wait_gather(slot):
            pltpu.make_async_copy(
                table_hbm.at[pl.ds(0, _CH)], rows_v.at[slot], gsem[slot]
            ).wait()

        def issue_writes(k, slot):
            pltpu.async_copy(
                rows_v.at[slot],
                out_hbm.at[pl.ds(out_off(k), _CH)],
                wsem[slot],
            )

        def drain_writes(slot):
            pltpu.make_async_copy(
                rows_v.at[slot], out_hbm.at[pl.ds(0, _CH)], wsem[slot]
            ).wait()

        issue_gather(0, 0)

        @pl.loop(0, steps, step=_NBUF)
        def _(t):
            for b in range(_NBUF):
                k = t + b
                nslot = (b + 1) % _NBUF

                @pl.when(k + 1 < steps)
                def _prefetch():
                    @pl.when(k + 1 >= _NBUF)
                    def _drain():
                        drain_writes(nslot)

                    issue_gather(k + 1, nslot)

                wait_gather(b)
                issue_writes(k, b)

        for b in range(_NBUF):
            drain_writes(b)

    return gather_kernel


def kernel(style_idx, table):
    batch, hist = style_idx.shape
    num_rows, embed_dim = table.shape
    pairs = hist * embed_dim // 128
    rc = pairs // _NCHUNKS
    bblk = _TC_BATCH_BLOCK
    jb = batch // bblk

    idx = style_idx.reshape(1, batch * hist).astype(jnp.int32)

    xt = None
    for ci in range(_NCHUNKS):
        g = _make_gather(batch, hist, embed_dim, ci * rc, rc, table.dtype)(
            table, idx
        )

        in3 = g.reshape(rc, batch, 128)

        def transpose_body(*refs):
            in_ref, out_ref = refs[0], refs[-1]
            for r in range(rc):
                out_ref[r] = in_ref[r].T

        operands = [in3] if xt is None else [in3, xt]
        in_specs = [
            pl.BlockSpec((rc, bblk, 128), lambda j: (0, j, 0)),
        ]
        if xt is not None:
            in_specs.append(pl.BlockSpec(memory_space=pltpu.MemorySpace.HBM))
        xt = pl.pallas_call(
            transpose_body,
            out_shape=jax.ShapeDtypeStruct((pairs, 128, batch), table.dtype),
            grid=(jb,),
            in_specs=in_specs,
            out_specs=pl.BlockSpec(
                (rc, 128, bblk), lambda j, _ci=ci: (_ci, 0, j)
            ),
            input_output_aliases={} if len(operands) == 1 else {1: 0},
            compiler_params=pltpu.CompilerParams(
                dimension_semantics=("arbitrary",)
            ),
        )(*operands)

    x3 = xt.reshape(hist, embed_dim, batch)
    return jnp.transpose(x3, (2, 0, 1))


# speedup vs baseline: 2.4937x; 1.0060x over previous
import functools

import jax
import jax.numpy as jnp
from jax import lax
from jax.experimental import pallas as pl
from jax.experimental.pallas import tpu as pltpu
from jax.experimental.pallas import tpu_sc as plsc

_NUM_WORKERS = 32
_NBUF = 2
_TC_BATCH_BLOCK = 4096
_NCHUNKS = 5
_CH = 256


def _make_gather(batch, hist, embed_dim, r_lo, rc, dtype):
    mesh = plsc.VectorSubcoreMesh(core_axis_name="c", subcore_axis_name="s")
    pb = batch // _NUM_WORKERS
    rows_r = 2 * pb
    spr = rows_r // _CH
    steps = rc * spr
    n_rows = rc * batch * 2

    @functools.partial(
        pl.kernel,
        out_type=jax.ShapeDtypeStruct((n_rows, embed_dim), dtype),
        mesh=mesh,
        compiler_params=pltpu.CompilerParams(
            use_tc_tiling_on_sc=False, needs_layout_passes=False
        ),
        scratch_types=[
            pltpu.VMEM((pb * hist,), jnp.int32),
            pltpu.VMEM((rc * rows_r,), jnp.int32),
            pltpu.VMEM((_NBUF, _CH, embed_dim), jnp.float32),
            pltpu.SemaphoreType.DMA,
            pltpu.SemaphoreType.DMA,
            pltpu.SemaphoreType.DMA,
            pltpu.SemaphoreType.DMA,
        ],
    )
    def gather_kernel(
        table_hbm, idx_hbm, out_hbm, idx_raw, idx_v, rows_v, g0, g1, w0, w1
    ):
        gsem = [g0, g1]
        wsem = [w0, w1]
        wid = lax.axis_index("s") * 2 + lax.axis_index("c")
        b0 = wid * pb

        pltpu.sync_copy(idx_hbm.at[0, pl.ds(b0 * hist, pb * hist)], idx_raw)

        lane = jnp.arange(16, dtype=jnp.int32)
        b_of = lane // 2
        hh_of = lane - 2 * b_of

        @pl.loop(0, rc * rows_r // 16)
        def _(gidx):
            r = gidx // (rows_r // 16)
            bg = gidx - r * (rows_r // 16)
            src = (bg * 8 + b_of) * hist + 2 * (r_lo + r) + hh_of
            idx_v[pl.ds(gidx * 16, 16)] = plsc.load_gather(idx_raw, [src])

        def out_off(k):
            r = k // spr
            sub = k - r * spr
            return r * batch * 2 + b0 * 2 + sub * _CH

        def issue_gather(k, slot):
            pltpu.async_copy(
                table_hbm.at[idx_v.at[pl.ds(k * _CH, _CH)]],
                rows_v.at[slot],
                gsem[slot],
            )

        def wait_gather(slot):
            pltpu.make_async_copy(
                table_hbm.at[pl.ds(0, _CH)], rows_v.at[slot], gsem[slot]
            ).wait()

        def issue_writes(k, slot):
            pltpu.async_copy(
                rows_v.at[slot],
                out_hbm.at[pl.ds(out_off(k), _CH)],
                wsem[slot],
            )

        def drain_writes(slot):
            pltpu.make_async_copy(
                rows_v.at[slot], out_hbm.at[pl.ds(0, _CH)], wsem[slot]
            ).wait()

        issue_gather(0, 0)

        @pl.loop(0, steps, step=_NBUF)
        def _(t):
            for b in range(_NBUF):
                k = t + b
                nslot = (b + 1) % _NBUF

                @pl.when(k + 1 < steps)
                def _prefetch():
                    @pl.when(k + 1 >= _NBUF)
                    def _drain():
                        drain_writes(nslot)

                    issue_gather(k + 1, nslot)

                wait_gather(b)
                issue_writes(k, b)

        for b in range(_NBUF):
            drain_writes(b)

    return gather_kernel


def kernel(style_idx, table):
    batch, hist = style_idx.shape
    num_rows, embed_dim = table.shape
    pairs = hist * embed_dim // 128
    rc = pairs // _NCHUNKS
    bblk = _TC_BATCH_BLOCK
    jb = batch // bblk

    idx = style_idx.reshape(1, batch * hist).astype(jnp.int32)

    xt = None
    for ci in range(_NCHUNKS):
        g = _make_gather(batch, hist, embed_dim, ci * rc, rc, table.dtype)(
            table, idx
        )

        in3 = g.reshape(rc, batch, 128)

        def transpose_body(*refs):
            in_ref, out_ref = refs[0], refs[-1]
            for r in range(rc):
                out_ref[r] = in_ref[r].T

        operands = [in3] if xt is None else [in3, xt]
        in_specs = [
            pl.BlockSpec((rc, bblk, 128), lambda j: (0, j, 0)),
        ]
        if xt is not None:
            in_specs.append(pl.BlockSpec(memory_space=pltpu.MemorySpace.HBM))
        xt = pl.pallas_call(
            transpose_body,
            out_shape=jax.ShapeDtypeStruct((pairs, 128, batch), table.dtype),
            grid=(jb,),
            in_specs=in_specs,
            out_specs=pl.BlockSpec(
                (rc, 128, bblk), lambda j, _ci=ci: (_ci, 0, j)
            ),
            input_output_aliases={} if len(operands) == 1 else {1: 0},
            compiler_params=pltpu.CompilerParams(
                dimension_semantics=("arbitrary",)
            ),
        )(*operands)

    x3 = xt.reshape(hist, embed_dim, batch)
    return jnp.transpose(x3, (2, 0, 1))


# speedup vs baseline: 2.4973x; 1.0014x over previous
import functools

import jax
import jax.numpy as jnp
from jax import lax
from jax.experimental import pallas as pl
from jax.experimental.pallas import tpu as pltpu
from jax.experimental.pallas import tpu_sc as plsc

_NUM_WORKERS = 32
_NBUF = 2
_TC_BATCH_BLOCK = 4096
_NCHUNKS = 5
_CH = 256


def _make_gather(batch, hist, embed_dim, r_lo, rc, dtype):
    mesh = plsc.VectorSubcoreMesh(core_axis_name="c", subcore_axis_name="s")
    pb = batch // _NUM_WORKERS
    rows_r = 2 * pb
    spr = rows_r // _CH
    steps = rc * spr
    n_rows = rc * batch * 2

    @functools.partial(
        pl.kernel,
        out_type=jax.ShapeDtypeStruct((n_rows, embed_dim), dtype),
        mesh=mesh,
        compiler_params=pltpu.CompilerParams(
            use_tc_tiling_on_sc=False, needs_layout_passes=False
        ),
        scratch_types=[
            pltpu.VMEM((pb * hist,), jnp.int32),
            pltpu.VMEM((rc * rows_r,), jnp.int32),
            pltpu.VMEM((_NBUF, _CH, embed_dim), jnp.float32),
            pltpu.SemaphoreType.DMA,
            pltpu.SemaphoreType.DMA,
            pltpu.SemaphoreType.DMA,
            pltpu.SemaphoreType.DMA,
        ],
    )
    def gather_kernel(
        table_hbm, idx_hbm, out_hbm, idx_raw, idx_v, rows_v, g0, g1, w0, w1
    ):
        gsem = [g0, g1]
        wsem = [w0, w1]
        wid = lax.axis_index("s") * 2 + lax.axis_index("c")
        b0 = wid * pb

        pltpu.sync_copy(idx_hbm.at[0, pl.ds(b0 * hist, pb * hist)], idx_raw)

        lane = jnp.arange(16, dtype=jnp.int32)
        b_of = lane // 2
        hh_of = lane - 2 * b_of

        @pl.loop(0, rc * rows_r // 16)
        def _(gidx):
            r = gidx // (rows_r // 16)
            bg = gidx - r * (rows_r // 16)
            src = (bg * 8 + b_of) * hist + 2 * (r_lo + r) + hh_of
            idx_v[pl.ds(gidx * 16, 16)] = plsc.load_gather(idx_raw, [src])

        def out_off(k):
            r = k // spr
            sub = k - r * spr
            return r * batch * 2 + b0 * 2 + sub * _CH

        def issue_gather(k, slot):
            pltpu.async_copy(
                table_hbm.at[idx_v.at[pl.ds(k * _CH, _CH)]],
                rows_v.at[slot],
                gsem[slot],
            )

        def wait_gather(slot):
            pltpu.make_async_copy(
                table_hbm.at[pl.ds(0, _CH)], rows_v.at[slot], gsem[slot]
            ).wait()

        def issue_writes(k, slot):
            pltpu.async_copy(
                rows_v.at[slot],
                out_hbm.at[pl.ds(out_off(k), _CH)],
                wsem[slot],
            )

        def drain_writes(slot):
            pltpu.make_async_copy(
                rows_v.at[slot], out_hbm.at[pl.ds(0, _CH)], wsem[slot]
            ).wait()

        issue_gather(0, 0)

        @pl.loop(0, steps, step=_NBUF)
        def _(t):
            for b in range(_NBUF):
                k = t + b
                nslot = (b + 1) % _NBUF

                @pl.when(k + 1 < steps)
                def _prefetch():
                    @pl.when(k + 1 >= _NBUF)
                    def _drain():
                        drain_writes(nslot)

                    issue_gather(k + 1, nslot)

                wait_gather(b)
                issue_writes(k, b)

        for b in range(_NBUF):
            drain_writes(b)

    return gather_kernel


def kernel(style_idx, table):
    batch, hist = style_idx.shape
    num_rows, embed_dim = table.shape
    pairs = hist * embed_dim // 128
    rc = pairs // _NCHUNKS
    bblk = _TC_BATCH_BLOCK
    jb = batch // bblk

    idx = style_idx.reshape(1, batch * hist).astype(jnp.int32)

    xt = None
    for ci in range(_NCHUNKS):
        g = _make_gather(batch, hist, embed_dim, ci * rc, rc, table.dtype)(
            table, idx
        )

        in3 = g.reshape(rc, batch, 128)

        def transpose_body(*refs):
            in_ref, out_ref = refs[0], refs[-1]
            for r in range(rc):
                out_ref[r] = in_ref[r].T

        operands = [in3] if xt is None else [in3, xt]
        in_specs = [
            pl.BlockSpec((rc, bblk, 128), lambda j: (0, j, 0)),
        ]
        if xt is not None:
            in_specs.append(pl.BlockSpec(memory_space=pltpu.MemorySpace.HBM))
        xt = pl.pallas_call(
            transpose_body,
            out_shape=jax.ShapeDtypeStruct((pairs, 128, batch), table.dtype),
            grid=(jb,),
            in_specs=in_specs,
            out_specs=pl.BlockSpec(
                (rc, 128, bblk), lambda j, _ci=ci: (_ci, 0, j)
            ),
            input_output_aliases={} if len(operands) == 1 else {1: 0},
            compiler_params=pltpu.CompilerParams(
                dimension_semantics=("parallel",)
            ),
        )(*operands)

    x3 = xt.reshape(hist, embed_dim, batch)
    return jnp.transpose(x3, (2, 0, 1))


# speedup vs baseline: 2.5350x; 1.0151x over previous
import functools

import jax
import jax.numpy as jnp
from jax import lax
from jax.experimental import pallas as pl
from jax.experimental.pallas import tpu as pltpu
from jax.experimental.pallas import tpu_sc as plsc

_NUM_WORKERS = 32
_NBUF = 2
_TC_BATCH_BLOCK = 1024
_NCHUNKS = 1
_CH = 256


def _make_gather(batch, hist, embed_dim, r_lo, rc, dtype):
    mesh = plsc.VectorSubcoreMesh(core_axis_name="c", subcore_axis_name="s")
    pb = batch // _NUM_WORKERS
    rows_r = 2 * pb
    spr = rows_r // _CH
    steps = rc * spr
    n_rows = rc * batch * 2

    @functools.partial(
        pl.kernel,
        out_type=jax.ShapeDtypeStruct((n_rows, embed_dim), dtype),
        mesh=mesh,
        compiler_params=pltpu.CompilerParams(
            use_tc_tiling_on_sc=False, needs_layout_passes=False
        ),
        scratch_types=[
            pltpu.VMEM((pb * hist,), jnp.int32),
            pltpu.VMEM((rc * rows_r,), jnp.int32),
            pltpu.VMEM((_NBUF, _CH, embed_dim), jnp.float32),
            pltpu.SemaphoreType.DMA,
            pltpu.SemaphoreType.DMA,
            pltpu.SemaphoreType.DMA,
            pltpu.SemaphoreType.DMA,
        ],
    )
    def gather_kernel(
        table_hbm, idx_hbm, out_hbm, idx_raw, idx_v, rows_v, g0, g1, w0, w1
    ):
        gsem = [g0, g1]
        wsem = [w0, w1]
        wid = lax.axis_index("s") * 2 + lax.axis_index("c")
        b0 = wid * pb

        pltpu.sync_copy(idx_hbm.at[0, pl.ds(b0 * hist, pb * hist)], idx_raw)

        lane = jnp.arange(16, dtype=jnp.int32)
        b_of = lane // 2
        hh_of = lane - 2 * b_of

        @pl.loop(0, rc * rows_r // 16)
        def _(gidx):
            r = gidx // (rows_r // 16)
            bg = gidx - r * (rows_r // 16)
            src = (bg * 8 + b_of) * hist + 2 * (r_lo + r) + hh_of
            idx_v[pl.ds(gidx * 16, 16)] = plsc.load_gather(idx_raw, [src])

        def out_off(k):
            r = k // spr
            sub = k - r * spr
            return r * batch * 2 + b0 * 2 + sub * _CH

        def issue_gather(k, slot):
            pltpu.async_copy(
                table_hbm.at[idx_v.at[pl.ds(k * _CH, _CH)]],
                rows_v.at[slot],
                gsem[slot],
            )

        def wait_gather(slot):
            pltpu.make_async_copy(
                table_hbm.at[pl.ds(0, _CH)], rows_v.at[slot], gsem[slot]
            ).wait()

        def issue_writes(k, slot):
            pltpu.async_copy(
                rows_v.at[slot],
                out_hbm.at[pl.ds(out_off(k), _CH)],
                wsem[slot],
            )

        def drain_writes(slot):
            pltpu.make_async_copy(
                rows_v.at[slot], out_hbm.at[pl.ds(0, _CH)], wsem[slot]
            ).wait()

        issue_gather(0, 0)

        @pl.loop(0, steps, step=_NBUF)
        def _(t):
            for b in range(_NBUF):
                k = t + b
                nslot = (b + 1) % _NBUF

                @pl.when(k + 1 < steps)
                def _prefetch():
                    @pl.when(k + 1 >= _NBUF)
                    def _drain():
                        drain_writes(nslot)

                    issue_gather(k + 1, nslot)

                wait_gather(b)
                issue_writes(k, b)

        for b in range(_NBUF):
            drain_writes(b)

    return gather_kernel


def kernel(style_idx, table):
    batch, hist = style_idx.shape
    num_rows, embed_dim = table.shape
    pairs = hist * embed_dim // 128
    rc = pairs // _NCHUNKS
    bblk = _TC_BATCH_BLOCK
    jb = batch // bblk

    idx = style_idx.reshape(1, batch * hist).astype(jnp.int32)

    xt = None
    for ci in range(_NCHUNKS):
        g = _make_gather(batch, hist, embed_dim, ci * rc, rc, table.dtype)(
            table, idx
        )

        in3 = g.reshape(rc, batch, 128)

        def transpose_body(*refs):
            in_ref, out_ref = refs[0], refs[-1]
            for r in range(rc):
                out_ref[r] = in_ref[r].T

        operands = [in3] if xt is None else [in3, xt]
        in_specs = [
            pl.BlockSpec((rc, bblk, 128), lambda j: (0, j, 0)),
        ]
        if xt is not None:
            in_specs.append(pl.BlockSpec(memory_space=pltpu.MemorySpace.HBM))
        xt = pl.pallas_call(
            transpose_body,
            out_shape=jax.ShapeDtypeStruct((pairs, 128, batch), table.dtype),
            grid=(jb,),
            in_specs=in_specs,
            out_specs=pl.BlockSpec(
                (rc, 128, bblk), lambda j, _ci=ci: (_ci, 0, j)
            ),
            input_output_aliases={} if len(operands) == 1 else {1: 0},
            compiler_params=pltpu.CompilerParams(
                dimension_semantics=("parallel",)
            ),
        )(*operands)

    x3 = xt.reshape(hist, embed_dim, batch)
    return jnp.transpose(x3, (2, 0, 1))


# speedup vs baseline: 2.5786x; 1.0172x over previous
import functools

import jax
import jax.numpy as jnp
from jax import lax
from jax.experimental import pallas as pl
from jax.experimental.pallas import tpu as pltpu
from jax.experimental.pallas import tpu_sc as plsc

_NUM_WORKERS = 32
_GROUP = 8
_NBUF = 2
_TC_BATCH_BLOCK = 512


def kernel(style_idx, table):
    batch, hist = style_idx.shape
    num_rows, embed_dim = table.shape
    n = batch * hist

    idx = style_idx.reshape(1, n).astype(jnp.int32)
    mesh = plsc.VectorSubcoreMesh(core_axis_name="c", subcore_axis_name="s")

    per_w = batch // _NUM_WORKERS
    ch = _GROUP * hist
    steps = per_w // _GROUP

    @functools.partial(
        pl.kernel,
        out_type=jax.ShapeDtypeStruct((n, embed_dim), table.dtype),
        mesh=mesh,
        compiler_params=pltpu.CompilerParams(use_tc_tiling_on_sc=False),
        scratch_types=[
            pltpu.VMEM((1, per_w * hist), jnp.int32),
            pltpu.VMEM((_NBUF, ch, embed_dim), jnp.float32),
            pltpu.SemaphoreType.DMA,
            pltpu.SemaphoreType.DMA,
            pltpu.SemaphoreType.DMA,
            pltpu.SemaphoreType.DMA,
        ],
    )
    def gather_kernel(table_hbm, idx_hbm, out_hbm, idx_v, rows_v, g0, g1, w0, w1):
        gsem = [g0, g1]
        wsem = [w0, w1]
        wid = lax.axis_index("s") * 2 + lax.axis_index("c")
        b0 = wid * per_w

        pltpu.sync_copy(idx_hbm.at[0, pl.ds(b0 * hist, per_w * hist)], idx_v.at[0])

        def issue_gather(k, slot):
            pltpu.async_copy(
                table_hbm.at[idx_v.at[0, pl.ds(k * ch, ch)]],
                rows_v.at[slot],
                gsem[slot],
            )

        def wait_gather(slot):
            pltpu.make_async_copy(
                table_hbm.at[pl.ds(0, ch)], rows_v.at[slot], gsem[slot]
            ).wait()

        def issue_writes(k, slot):
            pltpu.async_copy(
                rows_v.at[slot],
                out_hbm.at[pl.ds((b0 + k * _GROUP) * hist, ch)],
                wsem[slot],
            )

        def drain_writes(slot):
            pltpu.make_async_copy(
                rows_v.at[slot], out_hbm.at[pl.ds(0, ch)], wsem[slot]
            ).wait()

        issue_gather(0, 0)

        @pl.loop(0, steps, step=_NBUF)
        def _(t):
            for b in range(_NBUF):
                k = t + b
                nslot = (b + 1) % _NBUF

                @pl.when(k + 1 < steps)
                def _prefetch():
                    @pl.when(k + 1 >= _NBUF)
                    def _drain():
                        drain_writes(nslot)

                    issue_gather(k + 1, nslot)

                wait_gather(b)
                issue_writes(k, b)

        for b in range(_NBUF):
            drain_writes(b)

    gathered = gather_kernel(table, idx)

    row = hist * embed_dim
    rpb = row // 128
    in2d = gathered.reshape(n * embed_dim // 128, 128)

    bblk = _TC_BATCH_BLOCK

    def transpose_body(in_ref, out_ref):
        for r in range(rpb):
            out_ref[pl.ds(r * 128, 128), :] = in_ref[r::rpb, :].T

    xt = pl.pallas_call(
        transpose_body,
        out_shape=jax.ShapeDtypeStruct((row, batch), table.dtype),
        grid=(batch // bblk,),
        in_specs=[pl.BlockSpec((bblk * rpb, 128), lambda j: (j, 0))],
        out_specs=pl.BlockSpec((row, bblk), lambda j: (0, j)),
        compiler_params=pltpu.CompilerParams(
            dimension_semantics=("parallel",)
        ),
    )(in2d)

    x3 = xt.reshape(hist, embed_dim, batch)
    return jnp.transpose(x3, (2, 0, 1))


# speedup vs baseline: 2.5967x; 1.0070x over previous
import functools

import jax
import jax.numpy as jnp
from jax import lax
from jax.experimental import pallas as pl
from jax.experimental.pallas import tpu as pltpu
from jax.experimental.pallas import tpu_sc as plsc

_NUM_WORKERS = 32
_GROUP = 8
_NBUF = 2
_TC_BATCH_BLOCK = 1024


def kernel(style_idx, table):
    batch, hist = style_idx.shape
    num_rows, embed_dim = table.shape
    n = batch * hist

    idx = style_idx.reshape(1, n).astype(jnp.int32)
    mesh = plsc.VectorSubcoreMesh(core_axis_name="c", subcore_axis_name="s")

    per_w = batch // _NUM_WORKERS
    ch = _GROUP * hist
    steps = per_w // _GROUP

    @functools.partial(
        pl.kernel,
        out_type=jax.ShapeDtypeStruct((n, embed_dim), table.dtype),
        mesh=mesh,
        compiler_params=pltpu.CompilerParams(use_tc_tiling_on_sc=False),
        scratch_types=[
            pltpu.VMEM((1, per_w * hist), jnp.int32),
            pltpu.VMEM((_NBUF, ch, embed_dim), jnp.float32),
            pltpu.SemaphoreType.DMA,
            pltpu.SemaphoreType.DMA,
            pltpu.SemaphoreType.DMA,
            pltpu.SemaphoreType.DMA,
        ],
    )
    def gather_kernel(table_hbm, idx_hbm, out_hbm, idx_v, rows_v, g0, g1, w0, w1):
        gsem = [g0, g1]
        wsem = [w0, w1]
        wid = lax.axis_index("s") * 2 + lax.axis_index("c")
        b0 = wid * per_w

        pltpu.sync_copy(idx_hbm.at[0, pl.ds(b0 * hist, per_w * hist)], idx_v.at[0])

        def issue_gather(k, slot):
            pltpu.async_copy(
                table_hbm.at[idx_v.at[0, pl.ds(k * ch, ch)]],
                rows_v.at[slot],
                gsem[slot],
            )

        def wait_gather(slot):
            pltpu.make_async_copy(
                table_hbm.at[pl.ds(0, ch)], rows_v.at[slot], gsem[slot]
            ).wait()

        def issue_writes(k, slot):
            pltpu.async_copy(
                rows_v.at[slot],
                out_hbm.at[pl.ds((b0 + k * _GROUP) * hist, ch)],
                wsem[slot],
            )

        def drain_writes(slot):
            pltpu.make_async_copy(
                rows_v.at[slot], out_hbm.at[pl.ds(0, ch)], wsem[slot]
            ).wait()

        issue_gather(0, 0)

        @pl.loop(0, steps, step=_NBUF)
        def _(t):
            for b in range(_NBUF):
                k = t + b
                nslot = (b + 1) % _NBUF

                @pl.when(k + 1 < steps)
                def _prefetch():
                    @pl.when(k + 1 >= _NBUF)
                    def _drain():
                        drain_writes(nslot)

                    issue_gather(k + 1, nslot)

                wait_gather(b)
                issue_writes(k, b)

        for b in range(_NBUF):
            drain_writes(b)

    gathered = gather_kernel(table, idx)

    row = hist * embed_dim
    rpb = row // 128
    in2d = gathered.reshape(n * embed_dim // 128, 128)

    bblk = _TC_BATCH_BLOCK

    def transpose_body(in_ref, out_ref):
        for r in range(rpb):
            out_ref[pl.ds(r * 128, 128), :] = in_ref[r::rpb, :].T

    xt = pl.pallas_call(
        transpose_body,
        out_shape=jax.ShapeDtypeStruct((row, batch), table.dtype),
        grid=(batch // bblk,),
        in_specs=[pl.BlockSpec((bblk * rpb, 128), lambda j: (j, 0))],
        out_specs=pl.BlockSpec((row, bblk), lambda j: (0, j)),
        compiler_params=pltpu.CompilerParams(
            dimension_semantics=("parallel",)
        ),
    )(in2d)

    x3 = xt.reshape(hist, embed_dim, batch)
    return jnp.transpose(x3, (2, 0, 1))
